# Initial kernel scaffold; baseline (speedup 1.0000x reference)
import jax
import jax.numpy as jnp
from jax.experimental import pallas as pl


def kernel(x_nchw, w1, b1, w2, b2, fcw, fcb):
    raise NotImplementedError("write your pallas kernel here")


# speedup vs baseline: 1.1207x; 1.1207x over previous
import jax
import jax.numpy as jnp
from jax.experimental import pallas as pl
from jax.experimental.pallas import tpu as pltpu


def _make_csel(W):
    r = jax.lax.broadcasted_iota(jnp.int32, (W - 1, W // 2), 0)
    c = jax.lax.broadcasted_iota(jnp.int32, (W - 1, W // 2), 1)
    return (r == 2 * c).astype(jnp.float32)


def _pool2(a, csel):
    R, W = a.shape
    rs = a.reshape(R // 2, 2, W)
    rp = jnp.maximum(rs[:, 0, :], rs[:, 1, :])
    m = jnp.maximum(rp[:, :W - 1], rp[:, 1:])
    return jnp.dot(m, csel, preferred_element_type=jnp.float32)


def _fused_cnn(xp, fcw4, fcb, w1s, b1s, w2s, b2s):
    N, Hp, Wp = xp.shape
    H, W = Hp - 2, Wp - 2
    C1 = w1s.shape[0]
    C2 = w2s.shape[0]
    Ho, Wo = H // 2, W // 2
    Ho2, Wo2 = Ho // 2, Wo // 2
    ncls = fcw4.shape[0]

    def body(xp_ref, fcw_ref, fcb_ref, w1_ref, b1_ref, w2_ref, b2_ref,
             o_ref, h1s, h2s):
        csel1 = _make_csel(W)
        csel2 = _make_csel(Wo)

        h1s[...] = jnp.zeros((C1, Ho + 2, Wo + 2), jnp.float32)
        for r0 in range(0, H, 8):
            R = min(8, H - r0)
            accs = [None] * C1
            for dy in range(3):
                for dx in range(3):
                    tv = xp_ref[0, r0 + dy:r0 + dy + R, dx:dx + W]
                    for co in range(C1):
                        t = tv * w1_ref[co, dy * 3 + dx]
                        accs[co] = t if accs[co] is None else accs[co] + t
            for co in range(C1):
                p = _pool2(accs[co], csel1)
                h = jnp.maximum(p + b1_ref[co], 0.0)
                h1s[co, 1 + r0 // 2:1 + r0 // 2 + R // 2, 1:1 + Wo] = h

        for r0 in range(0, Ho, 8):
            R = min(8, Ho - r0)

            def ci_body(ci, accs, r0=r0, R=R):
                out = list(accs)
                for dy in range(3):
                    for dx in range(3):
                        tv = h1s[ci, r0 + dy:r0 + dy + R, dx:dx + Wo]
                        k = ci * 9 + dy * 3 + dx
                        for co in range(C2):
                            out[co] = out[co] + tv * w2_ref[co, k]
                return tuple(out)

            zero = jnp.zeros((R, Wo), jnp.float32)
            accs = jax.lax.fori_loop(0, C1, ci_body, (zero,) * C2)
            for co in range(C2):
                p = _pool2(accs[co], csel2)
                h2s[co, r0 // 2:r0 // 2 + R // 2, :] = (
                    jnp.maximum(p + b2_ref[co], 0.0))

        def fc_body(co, parts):
            v = h2s[co]
            return tuple(
                parts[cls] + jnp.sum(v * fcw_ref[cls, co],
                                     axis=0, keepdims=True)
                for cls in range(ncls))

        parts = jax.lax.fori_loop(
            0, C2, fc_body, (jnp.zeros((1, Wo2), jnp.float32),) * ncls)
        r = fcb_ref[0:1, :]
        iota = jax.lax.broadcasted_iota(jnp.int32, (1, ncls), 1)
        for cls in range(ncls):
            s = jnp.sum(parts[cls], axis=1, keepdims=True)
            r = r + jnp.where(iota == cls,
                              jnp.broadcast_to(s, (1, ncls)), 0.0)
        o_ref[...] = r.reshape(1, 1, ncls)

    return pl.pallas_call(
        body,
        out_shape=jax.ShapeDtypeStruct((N, 1, ncls), jnp.float32),
        grid=(N,),
        in_specs=[
            pl.BlockSpec((1, Hp, Wp), lambda n: (n, 0, 0)),
            pl.BlockSpec((ncls, C2, Ho2, Wo2), lambda n: (0, 0, 0, 0)),
            pl.BlockSpec((1, ncls), lambda n: (0, 0)),
            pl.BlockSpec(memory_space=pltpu.SMEM),
            pl.BlockSpec(memory_space=pltpu.SMEM),
            pl.BlockSpec(memory_space=pltpu.SMEM),
            pl.BlockSpec(memory_space=pltpu.SMEM),
        ],
        out_specs=pl.BlockSpec((1, 1, ncls), lambda n: (n, 0, 0)),
        scratch_shapes=[
            pltpu.VMEM((C1, Ho + 2, Wo + 2), jnp.float32),
            pltpu.VMEM((C2, Ho2, Wo2), jnp.float32),
        ],
        compiler_params=pltpu.CompilerParams(
            dimension_semantics=("parallel",)),
    )(xp, fcw4, fcb, w1s, b1s, w2s, b2s)


@jax.jit
def _forward(x_nchw, w1, b1, w2, b2, fcw, fcb):
    N, Cin, H, W = x_nchw.shape
    C1 = w1.shape[0]
    C2 = w2.shape[0]
    Ho2, Wo2 = H // 4, W // 4
    ncls = fcb.shape[-1]
    xp = jnp.pad(x_nchw.reshape(N, H, W).astype(jnp.float32),
                 ((0, 0), (1, 1), (1, 1)))
    w1s = w1.reshape(C1, Cin * 9).astype(jnp.float32)
    w2s = w2.reshape(C2, C1 * 9).astype(jnp.float32)
    b1s = b1.reshape(C1).astype(jnp.float32)
    b2s = b2.reshape(C2).astype(jnp.float32)
    K2 = C2 * Ho2 * Wo2
    fcw4 = fcw[:, :K2].reshape(ncls, C2, Ho2, Wo2).astype(jnp.float32)
    fcbr = fcb.reshape(1, ncls).astype(jnp.float32)
    out = _fused_cnn(xp, fcw4, fcbr, w1s, b1s, w2s, b2s)
    return out.reshape(N, ncls)


def kernel(x_nchw, w1, b1, w2, b2, fcw, fcb):
    return _forward(x_nchw, w1, b1, w2, b2, fcw, fcb)


# speedup vs baseline: 2.2520x; 2.0095x over previous
import jax
import jax.numpy as jnp
from jax.experimental import pallas as pl
from jax.experimental.pallas import tpu as pltpu


def _make_csel(W):
    r = jax.lax.broadcasted_iota(jnp.int32, (W - 1, W // 2), 0)
    c = jax.lax.broadcasted_iota(jnp.int32, (W - 1, W // 2), 1)
    return (r == 2 * c).astype(jnp.float32)


def _make_rsel(M):
    r = jax.lax.broadcasted_iota(jnp.int32, (M // 2, M - 1), 0)
    c = jax.lax.broadcasted_iota(jnp.int32, (M // 2, M - 1), 1)
    return (c == 2 * r).astype(jnp.float32)


def _pool2(a, rsel, csel):
    M, W = a.shape
    rowmax = jnp.maximum(a[:M - 1], a[1:])
    rp = jnp.dot(rsel, rowmax, preferred_element_type=jnp.float32)
    m = jnp.maximum(rp[:, :W - 1], rp[:, 1:])
    return jnp.dot(m, csel, preferred_element_type=jnp.float32)


def _fused_cnn(xp, fcw4, fcb, w1s, b1s, w2s, b2s):
    P, Hp, Wp2 = xp.shape
    H = Hp - 2
    S1 = Wp2 // 2
    W = S1 - 2
    C1 = w1s.shape[0]
    C2 = w2s.shape[0]
    Ho, Wo = H // 2, W // 2
    S2 = Wo + 2
    Ho2, Wo2 = Ho // 2, Wo // 2
    ncls = fcw4.shape[0]
    AW = Wp2 - 2
    AW2 = 2 * S2 - 2
    PW2 = AW2 // 2

    def body(xp_ref, fcw_ref, fcb_ref, w1_ref, b1_ref, w2_ref, b2_ref,
             o_ref, h1s, h2s):
        csel1 = _make_csel(AW)
        csel2 = _make_csel(AW2)

        h1s[...] = jnp.zeros((C1, Ho + 2, 2 * S2), jnp.float32)
        for r0 in range(0, H, 8):
            R = min(8, H - r0)
            accs = [None] * C1
            for dy in range(3):
                for dx in range(3):
                    tv = xp_ref[0, r0 + dy:r0 + dy + R, dx:dx + AW]
                    for co in range(C1):
                        t = tv * w1_ref[co, dy * 3 + dx]
                        accs[co] = t if accs[co] is None else accs[co] + t
            stk = jnp.concatenate(accs, axis=0)
            p = _pool2(stk, _make_rsel(C1 * R), csel1)
            q = R // 2
            for co in range(C1):
                h = jnp.maximum(p[co * q:co * q + q, :] + b1_ref[co], 0.0)
                r1 = 1 + r0 // 2
                h1s[co, r1:r1 + q, 1:1 + Wo] = h[:, 0:Wo]
                h1s[co, r1:r1 + q, S2 + 1:S2 + 1 + Wo] = (
                    h[:, Wo + 1:2 * Wo + 1])

        for r0 in range(0, Ho, 8):
            R = min(8, Ho - r0)

            def ci_body(ci, accs, r0=r0, R=R):
                out = list(accs)
                for dy in range(3):
                    for dx in range(3):
                        tv = h1s[ci, r0 + dy:r0 + dy + R, dx:dx + AW2]
                        k = ci * 9 + dy * 3 + dx
                        for co in range(C2):
                            out[co] = out[co] + tv * w2_ref[co, k]
                return tuple(out)

            zero = jnp.zeros((R, AW2), jnp.float32)
            accs = jax.lax.fori_loop(0, C1, ci_body, (zero,) * C2)
            stk = jnp.concatenate(accs, axis=0)
            p = _pool2(stk, _make_rsel(C2 * R), csel2)
            q = R // 2
            for co in range(C2):
                h2s[co, r0 // 2:r0 // 2 + q, :] = (
                    jnp.maximum(p[co * q:co * q + q, :] + b2_ref[co], 0.0))

        def fc_body(co, parts):
            pa, pb = parts
            va = h2s[co, :, 0:Wo2]
            vb = h2s[co, :, Wo2 + 1:2 * Wo2 + 1]
            na = tuple(
                pa[cls] + jnp.sum(va * fcw_ref[cls, co],
                                  axis=0, keepdims=True)
                for cls in range(ncls))
            nb = tuple(
                pb[cls] + jnp.sum(vb * fcw_ref[cls, co],
                                  axis=0, keepdims=True)
                for cls in range(ncls))
            return (na, nb)

        zp = (jnp.zeros((1, Wo2), jnp.float32),) * ncls
        pa, pb = jax.lax.fori_loop(0, C2, fc_body, (zp, zp))
        iota = jax.lax.broadcasted_iota(jnp.int32, (1, ncls), 1)
        for img, parts in enumerate((pa, pb)):
            r = fcb_ref[0:1, :]
            for cls in range(ncls):
                s = jnp.sum(parts[cls], axis=1, keepdims=True)
                r = r + jnp.where(iota == cls,
                                  jnp.broadcast_to(s, (1, ncls)), 0.0)
            o_ref[img:img + 1] = r.reshape(1, 1, ncls)

    return pl.pallas_call(
        body,
        out_shape=jax.ShapeDtypeStruct((2 * P, 1, ncls), jnp.float32),
        grid=(P,),
        in_specs=[
            pl.BlockSpec((1, Hp, Wp2), lambda n: (n, 0, 0)),
            pl.BlockSpec((ncls, C2, Ho2, Wo2), lambda n: (0, 0, 0, 0)),
            pl.BlockSpec((1, ncls), lambda n: (0, 0)),
            pl.BlockSpec(memory_space=pltpu.SMEM),
            pl.BlockSpec(memory_space=pltpu.SMEM),
            pl.BlockSpec(memory_space=pltpu.SMEM),
            pl.BlockSpec(memory_space=pltpu.SMEM),
        ],
        out_specs=pl.BlockSpec((2, 1, ncls), lambda n: (n, 0, 0)),
        scratch_shapes=[
            pltpu.VMEM((C1, Ho + 2, 2 * S2), jnp.float32),
            pltpu.VMEM((C2, Ho2, PW2), jnp.float32),
        ],
        compiler_params=pltpu.CompilerParams(
            dimension_semantics=("parallel",)),
    )(xp, fcw4, fcb, w1s, b1s, w2s, b2s)


@jax.jit
def _forward(x_nchw, w1, b1, w2, b2, fcw, fcb):
    N, Cin, H, W = x_nchw.shape
    C1 = w1.shape[0]
    C2 = w2.shape[0]
    Ho2, Wo2 = H // 4, W // 4
    ncls = fcb.shape[-1]
    xpair = jnp.pad(
        x_nchw.astype(jnp.float32).reshape(N // 2, 2, H, W),
        ((0, 0), (0, 0), (1, 1), (1, 1)))
    xp = xpair.transpose(0, 2, 1, 3).reshape(N // 2, H + 2, 2 * (W + 2))
    w1s = w1.reshape(C1, Cin * 9).astype(jnp.float32)
    w2s = w2.reshape(C2, C1 * 9).astype(jnp.float32)
    b1s = b1.reshape(C1).astype(jnp.float32)
    b2s = b2.reshape(C2).astype(jnp.float32)
    K2 = C2 * Ho2 * Wo2
    fcw4 = fcw[:, :K2].reshape(ncls, C2, Ho2, Wo2).astype(jnp.float32)
    fcbr = fcb.reshape(1, ncls).astype(jnp.float32)
    out = _fused_cnn(xp, fcw4, fcbr, w1s, b1s, w2s, b2s)
    return out.reshape(N, ncls)


def kernel(x_nchw, w1, b1, w2, b2, fcw, fcb):
    return _forward(x_nchw, w1, b1, w2, b2, fcw, fcb)
